# baseline (device time: 87153 ns/iter reference)
import jax
import jax.numpy as jnp
from jax import lax
from jax.experimental import pallas as pl
from jax.experimental.pallas import tpu as pltpu

N_Z = 4
C = 8


def kernel(x):
    m, n = x.shape
    rows = m // C

    def body(x_ref, out_ref, f_in, b_in, f_out, b_out,
             f_send, f_recv, b_send, b_recv):
        my_x = lax.axis_index("x")
        my_y = lax.axis_index("y")
        my_z = lax.axis_index("z")
        right = jnp.minimum(my_z + 1, N_Z - 1)
        left = jnp.maximum(my_z - 1, 0)
        is_first = my_z == 0
        is_last = my_z == N_Z - 1
        is_mid = jnp.logical_and(my_z > 0, my_z < N_Z - 1)

        barrier = pltpu.get_barrier_semaphore()
        for nbr in (left, right):
            pl.semaphore_signal(
                barrier, inc=1,
                device_id=(my_x, my_y, nbr),
                device_id_type=pl.DeviceIdType.MESH,
            )
        pl.semaphore_wait(barrier, 2)

        def xs(c):
            return pl.ds(c * rows, rows)

        def f_rdma(c, from_x):
            return pltpu.make_async_remote_copy(
                src_ref=x_ref.at[xs(c), :] if from_x else f_out.at[c],
                dst_ref=f_in.at[c],
                send_sem=f_send.at[c],
                recv_sem=f_recv.at[c],
                device_id=(my_x, my_y, right),
                device_id_type=pl.DeviceIdType.MESH,
            )

        def b_rdma(c, from_x):
            return pltpu.make_async_remote_copy(
                src_ref=x_ref.at[xs(c), :] if from_x else b_out.at[c],
                dst_ref=b_in.at[c],
                send_sem=b_send.at[c],
                recv_sem=b_recv.at[c],
                device_id=(my_x, my_y, left),
                device_id_type=pl.DeviceIdType.MESH,
            )

        @pl.when(is_first)
        def _():
            for c in range(C):
                f_rdma(c, True).start()
            for c in range(C):
                b_rdma(c, False).wait_recv()
                out_ref[xs(c), :] = x_ref[xs(c), :] + b_in[c, :, :]
            for c in range(C):
                f_rdma(c, True).wait_send()

        @pl.when(is_last)
        def _():
            for c in range(C):
                b_rdma(c, True).start()
            for c in range(C):
                f_rdma(c, False).wait_recv()
                out_ref[xs(c), :] = x_ref[xs(c), :] + f_in[c, :, :]
            for c in range(C):
                b_rdma(c, True).wait_send()

        @pl.when(is_mid)
        def _():
            for c in range(C):
                f_rdma(c, False).wait_recv()
                f_out[c, :, :] = f_in[c, :, :] + x_ref[xs(c), :]
                f_rdma(c, False).start()
                b_rdma(c, False).wait_recv()
                b_out[c, :, :] = b_in[c, :, :] + x_ref[xs(c), :]
                b_rdma(c, False).start()
                out_ref[xs(c), :] = f_out[c, :, :] + b_in[c, :, :]
            for c in range(C):
                f_rdma(c, False).wait_send()
                b_rdma(c, False).wait_send()

    return pl.pallas_call(
        body,
        out_shape=jax.ShapeDtypeStruct((m, n), x.dtype),
        in_specs=[pl.BlockSpec(memory_space=pltpu.VMEM)],
        out_specs=pl.BlockSpec(memory_space=pltpu.VMEM),
        scratch_shapes=[
            pltpu.VMEM((C, rows, n), x.dtype),
            pltpu.VMEM((C, rows, n), x.dtype),
            pltpu.VMEM((C, rows, n), x.dtype),
            pltpu.VMEM((C, rows, n), x.dtype),
            pltpu.SemaphoreType.DMA((C,)),
            pltpu.SemaphoreType.DMA((C,)),
            pltpu.SemaphoreType.DMA((C,)),
            pltpu.SemaphoreType.DMA((C,)),
        ],
        compiler_params=pltpu.CompilerParams(collective_id=0),
    )(x)
